# baseline (speedup 1.0000x reference)
import functools

import numpy as np

import jax
import jax.numpy as jnp
from jax import lax
from jax.experimental import pallas as pl
from jax.experimental.pallas import tpu as pltpu
from jax.experimental.pallas import tpu_sc as plsc

ROWS = 1024
N = 4096
M = 2 * N
L = 16
NC = 2
NS = 16
NW = NC * NS
RPW = ROWS // NW
NV = M // L
P = 2
PLANE = M // P
NCHUNK = L * P
T = PLANE // L
TSH = T.bit_length() - 1
RADIX = 256
PASSES = (0, 8, 16, 24)

_I32_MIN = np.int32(-(2**31))


def _to_key(v):
    b = lax.bitcast_convert_type(v, jnp.int32)
    m = lax.shift_right_arithmetic(b, 31)
    return lax.bitwise_xor(b, lax.bitwise_or(m, _I32_MIN))


def _phys(p):
    return lax.bitwise_or(
        lax.bitwise_and(p, np.int32(~(PLANE - 1))),
        lax.bitwise_or(
            lax.shift_left(lax.bitwise_and(p, T - 1), 4),
            lax.shift_right_logical(lax.bitwise_and(p, PLANE - 1), TSH)))


def _digit(k, shift):
    if shift:
        k = lax.shift_right_arithmetic(k, shift)
    return lax.bitwise_and(k, RADIX - 1)


def _sc_body(xs_hbm, xt_hbm, out_hbm,
             raw_s, raw_t, key_a, key_b, side_a, side_b,
             hist_n, hist_a, hist_b, accv):
    cid = lax.axis_index("c")
    sid = lax.axis_index("s")
    wid = cid * NS + sid
    lane = lax.iota(jnp.int32, L)
    ones = jnp.ones((L,), jnp.int32)
    zeros = jnp.zeros((L,), jnp.int32)
    hists = (hist_a, hist_b)

    def zero_n(i, _):
        hist_n[pl.ds(i * L, L)] = zeros
        return 0
    lax.fori_loop(0, RADIX * P, zero_n, 0, unroll=8)

    def prefix_pass(shift):
        def body(d, carry):
            base = d * (L * P)
            va = hist_n[pl.ds(base, L)]
            vb = hist_n[pl.ds(base + L, L)]
            csa = plsc.cumsum(va)
            csb = plsc.cumsum(vb)
            sa = jnp.sum(va)
            hist_a[pl.ds(d * L, L)] = csa - va + carry
            hist_b[pl.ds(d * L, L)] = csb - vb + (carry + sa)
            hist_n[pl.ds(base, L)] = zeros
            hist_n[pl.ds(base + L, L)] = zeros
            return carry + sa + jnp.sum(vb)
        lax.fori_loop(0, RADIX, body, jnp.int32(0), unroll=4)

    def permute_pass(inkey, inside, outkey, outside, shift, first, nshift,
                     last):
        def body(t, _):
            for j in range(P):
                k = inkey[pl.ds(j * PLANE + t * L, L)]
                if first:
                    side = jnp.where(lane < (L * P // 2 - j * L), ones, -ones)
                else:
                    side = inside[pl.ds(j * PLANE + t * L, L)]
                idx = _digit(k, shift) * L + lane
                hj = hists[j]
                off = plsc.load_gather(hj, [idx])
                plsc.store_scatter(hj, [idx], off + 1)
                if last:
                    plsc.store_scatter(outside, [off], side)
                else:
                    dest = _phys(off)
                    plsc.store_scatter(outkey, [dest], k)
                    plsc.store_scatter(outside, [dest], side)
                    idx2 = (_digit(k, nshift) * NCHUNK
                            + lax.shift_right_logical(off, TSH))
                    plsc.addupdate_scatter(hist_n, [idx2], ones)
            return 0
        lax.fori_loop(0, T, body, 0, unroll=4)

    def row_body(r, acc):
        row = wid * RPW + r
        pltpu.sync_copy(xs_hbm.at[row], raw_s)
        pltpu.sync_copy(xt_hbm.at[row], raw_t)

        def pre(raw, pbase):
            def body(u, _):
                v = raw[pl.ds(u * L, L)]
                p = pbase + u * L + lane
                k = _to_key(v)
                plsc.store_scatter(key_a, [_phys(p)], k)
                idx2 = (_digit(k, 0) * NCHUNK
                        + lax.shift_right_logical(p, TSH))
                plsc.addupdate_scatter(hist_n, [idx2], ones)
                return 0
            lax.fori_loop(0, N // L, body, 0, unroll=8)
        pre(raw_s, 0)
        pre(raw_t, N)

        prefix_pass(0)
        permute_pass(key_a, None, key_b, side_b, 0, True, 8, False)
        prefix_pass(8)
        permute_pass(key_b, side_b, key_a, side_a, 8, False, 16, False)
        prefix_pass(16)
        permute_pass(key_a, side_a, key_b, side_b, 16, False, 24, False)
        prefix_pass(24)
        permute_pass(key_b, side_b, None, side_a, 24, False, 0, True)

        def walk(i, carry):
            d0, mx, mn = carry
            s = side_a[pl.ds(i * L, L)]
            d = plsc.cumsum(s) + d0
            return (d0 + jnp.sum(s), jnp.maximum(mx, d), jnp.minimum(mn, d))
        d0, mx, mn = lax.fori_loop(
            0, NV, walk, (jnp.int32(0), zeros, zeros), unroll=4)
        dn = jnp.maximum(jnp.max(mx), -jnp.min(mn))

        f = dn.astype(jnp.float32)
        e = (f * f) * jnp.float32(-1.0 / N)
        val = jnp.float32(2.0) * jnp.exp(lax.broadcast(e, (L,)))
        return acc + jnp.where(lane < 1, val, jnp.float32(0.0))

    acc = lax.fori_loop(0, RPW, row_body, jnp.zeros((L,), jnp.float32))
    accv[...] = acc
    pltpu.sync_copy(accv, out_hbm.at[wid])


def kernel(xs, xt, alpha):
    del alpha
    mesh = plsc.VectorSubcoreMesh(
        core_axis_name="c", subcore_axis_name="s",
        num_cores=NC, num_subcores=NS)
    out = pl.kernel(
        _sc_body,
        out_type=jax.ShapeDtypeStruct((NW, L), jnp.float32),
        mesh=mesh,
        compiler_params=pltpu.CompilerParams(needs_layout_passes=False),
        scratch_types=[
            pltpu.VMEM((N,), jnp.float32),
            pltpu.VMEM((N,), jnp.float32),
            pltpu.VMEM((M,), jnp.int32),
            pltpu.VMEM((M,), jnp.int32),
            pltpu.VMEM((M,), jnp.int32),
            pltpu.VMEM((M,), jnp.int32),
            pltpu.VMEM((RADIX * L * P,), jnp.int32),
            pltpu.VMEM((RADIX * L,), jnp.int32),
            pltpu.VMEM((RADIX * L,), jnp.int32),
            pltpu.VMEM((L,), jnp.float32),
        ],
    )(xs, xt)
    return jnp.sum(out) / ROWS


# speedup vs baseline: 1.0629x; 1.0629x over previous
import functools

import numpy as np

import jax
import jax.numpy as jnp
from jax import lax
from jax.experimental import pallas as pl
from jax.experimental.pallas import tpu as pltpu
from jax.experimental.pallas import tpu_sc as plsc

ROWS = 1024
N = 4096
M = 2 * N
L = 16
NC = 2
NS = 16
NW = NC * NS
RPW = ROWS // NW
NV = M // L
P = 2
PLANE = M // P
NCHUNK = L * P
T = PLANE // L
TSH = T.bit_length() - 1
RADIX = 256

_I32_MIN = np.int32(-(2**31))


def _to_key(v):
    b = lax.bitcast_convert_type(v, jnp.int32)
    m = lax.shift_right_arithmetic(b, 31)
    return lax.bitwise_xor(b, lax.bitwise_or(m, _I32_MIN))


def _phys(p):
    return lax.bitwise_or(
        lax.bitwise_and(p, np.int32(~(PLANE - 1))),
        lax.bitwise_or(
            lax.shift_left(lax.bitwise_and(p, T - 1), 4),
            lax.shift_right_logical(lax.bitwise_and(p, PLANE - 1), TSH)))


def _digit(k, shift):
    if shift:
        k = lax.shift_right_arithmetic(k, shift)
    return lax.bitwise_and(k, RADIX - 1)


def _sc_body(xs_hbm, xt_hbm, out_hbm,
             raw_s, raw_t, key_a, key_b, hist_n, hist_a, hist_b, accv):
    cid = lax.axis_index("c")
    sid = lax.axis_index("s")
    wid = cid * NS + sid
    lane = lax.iota(jnp.int32, L)
    ones = jnp.ones((L,), jnp.int32)
    zeros = jnp.zeros((L,), jnp.int32)
    hists = (hist_a, hist_b)

    def zero_n(i, _):
        hist_n[pl.ds(i * L, L)] = zeros
        return 0
    lax.fori_loop(0, RADIX * P, zero_n, 0, unroll=8)

    def hist1_pass():
        def body(t, _):
            for j in range(P):
                k = key_a[pl.ds(j * PLANE + t * L, L)]
                idx = _digit(k, 0) * NCHUNK + (j * L + lane)
                plsc.addupdate_scatter(hist_n, [idx], ones)
            return 0
        lax.fori_loop(0, T, body, 0, unroll=8)

    def prefix_pass():
        def body(d, carry):
            base = d * NCHUNK
            va = hist_n[pl.ds(base, L)]
            vb = hist_n[pl.ds(base + L, L)]
            csa = plsc.cumsum(va)
            csb = plsc.cumsum(vb)
            sa = jnp.sum(va)
            hist_a[pl.ds(d * L, L)] = csa - va + carry
            hist_b[pl.ds(d * L, L)] = csb - vb + (carry + sa)
            hist_n[pl.ds(base, L)] = zeros
            hist_n[pl.ds(base + L, L)] = zeros
            return carry + sa + jnp.sum(vb)
        lax.fori_loop(0, RADIX, body, jnp.int32(0), unroll=4)

    def permute_pass(inkey, outkey, shift, first, nshift, last):
        def body(t, _):
            for j in range(P):
                k = inkey[pl.ds(j * PLANE + t * L, L)]
                if first:
                    side01 = jnp.where(lane < (L * P // 2 - j * L),
                                       ones, zeros)
                    kout = lax.bitwise_or(
                        lax.bitwise_and(k, np.int32(~255)), side01)
                else:
                    kout = k
                idx = _digit(k, shift) * L + lane
                hj = hists[j]
                off = plsc.load_gather(hj, [idx])
                plsc.store_scatter(hj, [idx], off + 1)
                if last:
                    plsc.store_scatter(outkey, [off], kout)
                else:
                    plsc.store_scatter(outkey, [_phys(off)], kout)
                    idx2 = (_digit(k, nshift) * NCHUNK
                            + lax.shift_right_logical(off, TSH))
                    plsc.addupdate_scatter(hist_n, [idx2], ones)
            return 0
        lax.fori_loop(0, T, body, 0, unroll=4)

    def row_body(r, acc):
        row = wid * RPW + r
        pltpu.sync_copy(xs_hbm.at[row], raw_s)
        pltpu.sync_copy(xt_hbm.at[row], raw_t)

        def pre(raw, pbase):
            def body(u, _):
                v = raw[pl.ds(u * L, L)]
                p = pbase + u * L + lane
                plsc.store_scatter(key_a, [_phys(p)], _to_key(v))
                return 0
            lax.fori_loop(0, N // L, body, 0, unroll=8)
        pre(raw_s, 0)
        pre(raw_t, N)

        hist1_pass()
        prefix_pass()
        permute_pass(key_a, key_b, 0, True, 8, False)
        prefix_pass()
        permute_pass(key_b, key_a, 8, False, 16, False)
        prefix_pass()
        permute_pass(key_a, key_b, 16, False, 24, False)
        prefix_pass()
        permute_pass(key_b, key_a, 24, False, 0, True)

        def walk(i, carry):
            d0, mx, mn = carry
            k = key_a[pl.ds(i * L, L)]
            s = lax.shift_left(lax.bitwise_and(k, 1), 1) - 1
            d = plsc.cumsum(s) + d0
            return (d0 + jnp.sum(s), jnp.maximum(mx, d), jnp.minimum(mn, d))
        d0, mx, mn = lax.fori_loop(
            0, NV, walk, (jnp.int32(0), zeros, zeros), unroll=4)
        dn = jnp.maximum(jnp.max(mx), -jnp.min(mn))

        f = dn.astype(jnp.float32)
        e = (f * f) * jnp.float32(-1.0 / N)
        val = jnp.float32(2.0) * jnp.exp(lax.broadcast(e, (L,)))
        return acc + jnp.where(lane < 1, val, jnp.float32(0.0))

    acc = lax.fori_loop(0, RPW, row_body, jnp.zeros((L,), jnp.float32))
    accv[...] = acc
    pltpu.sync_copy(accv, out_hbm.at[wid])


def kernel(xs, xt, alpha):
    del alpha
    mesh = plsc.VectorSubcoreMesh(
        core_axis_name="c", subcore_axis_name="s",
        num_cores=NC, num_subcores=NS)
    out = pl.kernel(
        _sc_body,
        out_type=jax.ShapeDtypeStruct((NW, L), jnp.float32),
        mesh=mesh,
        compiler_params=pltpu.CompilerParams(needs_layout_passes=False),
        scratch_types=[
            pltpu.VMEM((N,), jnp.float32),
            pltpu.VMEM((N,), jnp.float32),
            pltpu.VMEM((M,), jnp.int32),
            pltpu.VMEM((M,), jnp.int32),
            pltpu.VMEM((RADIX * NCHUNK,), jnp.int32),
            pltpu.VMEM((RADIX * L,), jnp.int32),
            pltpu.VMEM((RADIX * L,), jnp.int32),
            pltpu.VMEM((L,), jnp.float32),
        ],
    )(xs, xt)
    return jnp.sum(out) / ROWS
